# baseline (speedup 1.0000x reference)
import jax
import jax.numpy as jnp
from jax.experimental import pallas as pl

_N_E = 512
_C = 32


def _vq_body(z_ref, emb_ref, out_ref):
    zb = z_ref[0]
    emb = emb_ref[...]
    e_sq = jnp.sum(emb * emb, axis=1)
    z_sq = jnp.sum(zb * zb, axis=0)
    ez2 = jax.lax.dot_general(emb + emb, zb, (((1,), (0,)), ((), ())),
                              preferred_element_type=jnp.float32)
    dist = (z_sq[None, :] + e_sq[:, None]) - ez2
    m = jnp.min(dist, axis=0)
    iota = jax.lax.broadcasted_iota(jnp.int32, dist.shape, 0)
    idx = jnp.min(jnp.where(dist == m[None, :], iota, _N_E), axis=0)
    onehot = (iota == idx[None, :]).astype(jnp.float32)
    zq = jax.lax.dot_general(emb, onehot, (((0,), (0,)), ((), ())),
                             preferred_element_type=jnp.float32)
    out_ref[0] = zq


def kernel(z, emb_weight):
    bs, c, h, w = z.shape
    s = h * w
    ts = 2048
    zf = z.reshape(bs, c, s)
    out = pl.pallas_call(
        _vq_body,
        grid=(bs, s // ts),
        in_specs=[
            pl.BlockSpec((1, c, ts), lambda i, j: (i, 0, j)),
            pl.BlockSpec((_N_E, _C), lambda i, j: (0, 0)),
        ],
        out_specs=pl.BlockSpec((1, c, ts), lambda i, j: (i, 0, j)),
        out_shape=jax.ShapeDtypeStruct((bs, c, s), jnp.float32),
    )(zf, emb_weight)
    return out.reshape(bs, c, h, w)


# speedup vs baseline: 1.3609x; 1.3609x over previous
import jax
import jax.numpy as jnp
from jax.experimental import pallas as pl

_N_E = 512
_C = 32


def _vq_body(z_ref, emb_ref, out_ref):
    th = z_ref.shape[2]
    ts = th * z_ref.shape[3]
    zb = z_ref[0].reshape(_C, ts)
    emb = emb_ref[...]
    e_sq = jnp.sum(emb * emb, axis=1)
    z_sq = jnp.sum(zb * zb, axis=0)
    ez2 = jax.lax.dot_general(emb + emb, zb, (((1,), (0,)), ((), ())),
                              preferred_element_type=jnp.float32)
    dist = (z_sq[None, :] + e_sq[:, None]) - ez2
    m = jnp.min(dist, axis=0)
    iota = jax.lax.broadcasted_iota(jnp.int32, dist.shape, 0)
    idx = jnp.min(jnp.where(dist == m[None, :], iota, _N_E), axis=0)
    onehot = (iota == idx[None, :]).astype(jnp.float32)
    zq = jax.lax.dot_general(emb, onehot, (((0,), (0,)), ((), ())),
                             preferred_element_type=jnp.float32)
    out_ref[0] = zq.reshape(_C, th, z_ref.shape[3])


def kernel(z, emb_weight):
    bs, c, h, w = z.shape
    th = 32
    out = pl.pallas_call(
        _vq_body,
        grid=(bs, h // th),
        in_specs=[
            pl.BlockSpec((1, c, th, w), lambda i, j: (i, 0, j, 0)),
            pl.BlockSpec((_N_E, _C), lambda i, j: (0, 0)),
        ],
        out_specs=pl.BlockSpec((1, c, th, w), lambda i, j: (i, 0, j, 0)),
        out_shape=jax.ShapeDtypeStruct((bs, c, h, w), jnp.float32),
    )(z, emb_weight)
    return out
